# baseline (speedup 1.0000x reference)
import jax
import jax.numpy as jnp
from jax.experimental import pallas as pl
from jax.experimental.pallas import tpu as pltpu

_CH = 1280
_NPAD = 10240


def _chunks(n):
    sizes = []
    off = 0
    while off < n:
        sizes.append(min(_CH, n - off))
        off += _CH
    return sizes


def _mlp_body(x_hbm, w1_ref, b1_ref, w2_ref, out_hbm,
              xbuf, obuf, in_sem, out_sem):
    n = x_hbm.shape[0]
    sizes = _chunks(n)

    def in_copy(i, sz):
        return pltpu.make_async_copy(
            x_hbm.at[pl.ds(i * _CH, sz), :], xbuf.at[i, pl.ds(0, sz), :],
            in_sem.at[i])

    def out_copy(i):
        return pltpu.make_async_copy(
            obuf.at[i], out_hbm.at[:, pl.ds(i * _CH, _CH)], out_sem.at[i])

    for i, sz in enumerate(sizes):
        in_copy(i, sz).start()
    for i, sz in enumerate(sizes):
        in_copy(i, sz).wait()
        h = jnp.dot(xbuf[i, :sz, :], w1_ref[:],
                    preferred_element_type=jnp.float32)
        h = jnp.maximum(h + b1_ref[:], 0.0)
        obuf[i, :, pl.ds(0, sz)] = jax.lax.dot_general(
            w2_ref[:], h, (((0,), (1,)), ((), ())),
            preferred_element_type=jnp.float32)
        out_copy(i).start()
    for i in range(len(sizes)):
        out_copy(i).wait()


def kernel(x, edge_index, W1, b1, W2, b2):
    n, d = x.shape
    hid = W1.shape[1]
    ncls = W2.shape[1]
    b1r = b1.reshape(1, hid)
    nslab = _NPAD // _CH
    out_t = pl.pallas_call(
        _mlp_body,
        grid=(1,),
        in_specs=[
            pl.BlockSpec(memory_space=pl.ANY),
            pl.BlockSpec((d, hid), lambda i: (0, 0)),
            pl.BlockSpec((1, hid), lambda i: (0, 0)),
            pl.BlockSpec((hid, ncls), lambda i: (0, 0)),
        ],
        out_specs=pl.BlockSpec(memory_space=pl.ANY),
        out_shape=jax.ShapeDtypeStruct((ncls, _NPAD), jnp.float32),
        scratch_shapes=[
            pltpu.VMEM((nslab, _CH, d), jnp.float32),
            pltpu.VMEM((nslab, ncls, _CH), jnp.float32),
            pltpu.SemaphoreType.DMA((nslab,)),
            pltpu.SemaphoreType.DMA((nslab,)),
        ],
    )(x, W1, b1r, W2)
    return out_t[:, :n].T + b2


# speedup vs baseline: 1.0962x; 1.0962x over previous
import jax
import jax.numpy as jnp
from jax.experimental import pallas as pl
from jax.experimental.pallas import tpu as pltpu

_CH = 3456
_NPAD = 10368


def _chunks(n):
    sizes = []
    off = 0
    while off < n:
        sizes.append(min(_CH, n - off))
        off += _CH
    return sizes


def _mlp_body(x_hbm, w1_ref, b1_ref, w2_ref, out_hbm,
              xbuf, obuf, in_sem, out_sem):
    n = x_hbm.shape[0]
    sizes = _chunks(n)

    def in_copy(i, sz):
        return pltpu.make_async_copy(
            x_hbm.at[pl.ds(i * _CH, sz), :], xbuf.at[i, pl.ds(0, sz), :],
            in_sem.at[i])

    def out_copy(i):
        return pltpu.make_async_copy(
            obuf.at[i], out_hbm.at[:, pl.ds(i * _CH, _CH)], out_sem.at[i])

    for i, sz in enumerate(sizes):
        in_copy(i, sz).start()
    for i, sz in enumerate(sizes):
        in_copy(i, sz).wait()
        h = jnp.dot(xbuf[i, :sz, :], w1_ref[:],
                    preferred_element_type=jnp.float32)
        h = jnp.maximum(h + b1_ref[:], 0.0)
        obuf[i, :, pl.ds(0, sz)] = jax.lax.dot_general(
            w2_ref[:], h, (((0,), (1,)), ((), ())),
            preferred_element_type=jnp.float32)
        out_copy(i).start()
    for i in range(len(sizes)):
        out_copy(i).wait()


def kernel(x, edge_index, W1, b1, W2, b2):
    n, d = x.shape
    hid = W1.shape[1]
    ncls = W2.shape[1]
    b1r = b1.reshape(1, hid)
    nslab = _NPAD // _CH
    out_t = pl.pallas_call(
        _mlp_body,
        grid=(1,),
        in_specs=[
            pl.BlockSpec(memory_space=pl.ANY),
            pl.BlockSpec((d, hid), lambda i: (0, 0)),
            pl.BlockSpec((1, hid), lambda i: (0, 0)),
            pl.BlockSpec((hid, ncls), lambda i: (0, 0)),
        ],
        out_specs=pl.BlockSpec(memory_space=pl.ANY),
        out_shape=jax.ShapeDtypeStruct((ncls, _NPAD), jnp.float32),
        scratch_shapes=[
            pltpu.VMEM((nslab, _CH, d), jnp.float32),
            pltpu.VMEM((nslab, ncls, _CH), jnp.float32),
            pltpu.SemaphoreType.DMA((nslab,)),
            pltpu.SemaphoreType.DMA((nslab,)),
        ],
    )(x, W1, b1r, W2)
    return out_t[:, :n].T + b2
